# baseline (speedup 1.0000x reference)
import jax
import jax.numpy as jnp
from jax import lax
from jax.experimental import pallas as pl
from jax.experimental.pallas import tpu as pltpu
from jax.experimental.pallas import tpu_sc as plsc

_N_E = 1024
_BETA = 0.25
_BLOCK = 2048
_NCHUNKS = 2

_NC = 2
_NS = 16
_NW = _NC * _NS
_CHUNK = 128
_STAGE = 4


def _vq_body(z_ref, cb_ref, idx_ref, acc_ref):
    i = pl.program_id(0)
    z = z_ref[...]
    cb = cb_ref[...]
    z2 = jnp.sum(z * z, axis=1, keepdims=True)
    e2 = jnp.sum(cb * cb, axis=1, keepdims=True).T
    mm2 = lax.dot_general(z, cb + cb, (((1,), (1,)), ((), ())),
                          preferred_element_type=jnp.float32)
    dist = (z2 + e2) - mm2
    mind = jnp.min(dist, axis=1, keepdims=True)
    col = lax.broadcasted_iota(jnp.int32, dist.shape, 1)
    idx = jnp.min(jnp.where(dist == mind, col, _N_E), axis=1)
    idx_ref[...] = idx.reshape(idx_ref.shape)

    @pl.when(i == 0)
    def _init():
        acc_ref[...] = jnp.zeros_like(acc_ref)

    acc_ref[...] += jnp.sum(mind).reshape(1, 1)


def _gather_body(cbp_hbm, idx_hbm, out_hbm, idx_v, rows_v, sem):
    wid = lax.axis_index("s") * _NC + lax.axis_index("c")
    nch = idx_hbm.shape[0] // _NW
    bpw = nch * _CHUNK
    stage_rows = _STAGE * _CHUNK
    pltpu.sync_copy(idx_hbm.at[pl.ds(wid * nch, nch)], idx_v)
    for p in range(nch // _STAGE):
        copies = [pltpu.async_copy(cbp_hbm.at[idx_v.at[p * _STAGE + j]],
                                   rows_v.at[pl.ds(j * _CHUNK, _CHUNK)], sem)
                  for j in range(_STAGE)]
        for c in copies:
            c.wait()
        pltpu.sync_copy(rows_v,
                        out_hbm.at[pl.ds(wid * bpw + p * stage_rows, stage_rows)])


def _vq_chunk(z, codebook, cbp, c, nc):
    d = z.shape[1]
    n_e = codebook.shape[0]
    nb = nc // _BLOCK
    base = c * nb
    idx3, acc = pl.pallas_call(
        _vq_body,
        grid=(nb,),
        in_specs=[
            pl.BlockSpec((_BLOCK, d), lambda i: (base + i, 0)),
            pl.BlockSpec((n_e, d), lambda i: (0, 0)),
        ],
        out_specs=[
            pl.BlockSpec((1, 1, _BLOCK), lambda i: (i, 0, 0)),
            pl.BlockSpec((1, 1), lambda i: (0, 0)),
        ],
        out_shape=[
            jax.ShapeDtypeStruct((nb, 1, _BLOCK), jnp.int32),
            jax.ShapeDtypeStruct((1, 1), jnp.float32),
        ],
    )(z, codebook)
    indices = idx3.reshape(nc)

    mesh = plsc.VectorSubcoreMesh(core_axis_name="c", subcore_axis_name="s")
    gather = pl.kernel(
        _gather_body,
        mesh=mesh,
        out_type=jax.ShapeDtypeStruct((nc, 128), jnp.float32),
        scratch_types=[
            pltpu.VMEM((nc // _NW // _CHUNK, _CHUNK), jnp.int32),
            pltpu.VMEM((_STAGE * _CHUNK, 128), jnp.float32),
            pltpu.SemaphoreType.DMA,
        ],
    )
    zqp = gather(cbp, indices.reshape(nc // _CHUNK, _CHUNK))
    return zqp, indices, acc


@jax.jit
def kernel(z, codebook):
    n, d = z.shape
    n_e = codebook.shape[0]
    cbp = jnp.concatenate(
        [codebook, jnp.zeros((n_e, 128 - d), jnp.float32)], axis=1)
    nc = n // _NCHUNKS
    zqps, idxs, accs = [], [], []
    for c in range(_NCHUNKS):
        zqp, idx_c, acc_c = _vq_chunk(z, codebook, cbp, c, nc)
        zqps.append(zqp)
        idxs.append(idx_c)
        accs.append(acc_c[0, 0])
    zq = jnp.concatenate(zqps, axis=0)[:, :d]
    indices = jnp.concatenate(idxs, axis=0)
    total = accs[0]
    for a in accs[1:]:
        total = total + a
    mean_sq = total / (n * d)
    loss = mean_sq + _BETA * mean_sq
    return (zq, loss, indices)


# speedup vs baseline: 1.1002x; 1.1002x over previous
import jax
import jax.numpy as jnp
from jax import lax
from jax.experimental import pallas as pl
from jax.experimental.pallas import tpu as pltpu
from jax.experimental.pallas import tpu_sc as plsc

_N_E = 1024
_BETA = 0.25
_BLOCK = 2048
_NCHUNKS = 4

_NC = 2
_NS = 16
_NW = _NC * _NS
_CHUNK = 128


def _vq_body(z_ref, cb2_ref, e2_ref, idx_ref, acc_ref):
    i = pl.program_id(0)
    z = z_ref[...]
    cb2 = cb2_ref[...]
    e2 = e2_ref[...]
    z2 = jnp.sum(z * z, axis=1, keepdims=True)
    mm2 = lax.dot_general(z, cb2, (((1,), (1,)), ((), ())),
                          preferred_element_type=jnp.float32)
    dist = (z2 + e2) - mm2
    mind = jnp.min(dist, axis=1, keepdims=True)
    col = lax.broadcasted_iota(jnp.int32, dist.shape, 1)
    idx = jnp.min(jnp.where(dist == mind, col, _N_E), axis=1)
    idx_ref[...] = idx.reshape(idx_ref.shape)

    @pl.when(i == 0)
    def _init():
        acc_ref[...] = jnp.zeros_like(acc_ref)

    acc_ref[...] += jnp.sum(mind).reshape(1, 1)


def _gather_body(cbp_hbm, idx_hbm, out_hbm, idx_v, rows_v, sem):
    wid = lax.axis_index("s") * _NC + lax.axis_index("c")
    nch = idx_hbm.shape[0] // _NW
    bpw = nch * _CHUNK
    pltpu.sync_copy(idx_hbm.at[pl.ds(wid * nch, nch)], idx_v)
    copies = [pltpu.async_copy(cbp_hbm.at[idx_v.at[j]],
                               rows_v.at[pl.ds(j * _CHUNK, _CHUNK)], sem)
              for j in range(nch)]
    for c in copies:
        c.wait()
    pltpu.sync_copy(rows_v, out_hbm.at[pl.ds(wid * bpw, bpw)])


@jax.jit
def kernel(z, codebook):
    n, d = z.shape
    n_e = codebook.shape[0]
    nrow = _BLOCK // _CHUNK
    nb = n // _BLOCK
    nbc = nb // _NCHUNKS
    nc = n // _NCHUNKS
    nch = nc // _NW // _CHUNK

    cb2 = codebook + codebook
    e2 = jnp.sum(codebook ** 2, axis=1).reshape(1, n_e)
    cbp = jnp.concatenate(
        [codebook, jnp.zeros((n_e, 128 - d), jnp.float32)], axis=1)
    mesh = plsc.VectorSubcoreMesh(core_axis_name="c", subcore_axis_name="s")

    gather = pl.kernel(
        _gather_body,
        mesh=mesh,
        out_type=jax.ShapeDtypeStruct((nc, 128), jnp.float32),
        scratch_types=[
            pltpu.VMEM((nch, _CHUNK), jnp.int32),
            pltpu.VMEM((nch * _CHUNK, 128), jnp.float32),
            pltpu.SemaphoreType.DMA,
        ],
    )

    zqps, idxs, accs = [], [], []
    for c in range(_NCHUNKS):
        base = c * nbc
        idx2d, acc = pl.pallas_call(
            _vq_body,
            grid=(nbc,),
            in_specs=[
                pl.BlockSpec((_BLOCK, d), lambda i, b=base: (b + i, 0)),
                pl.BlockSpec((n_e, d), lambda i: (0, 0)),
                pl.BlockSpec((1, n_e), lambda i: (0, 0)),
            ],
            out_specs=[
                pl.BlockSpec((nrow, _CHUNK), lambda i: (i, 0)),
                pl.BlockSpec((1, 1), lambda i: (0, 0)),
            ],
            out_shape=[
                jax.ShapeDtypeStruct((nc // _CHUNK, _CHUNK), jnp.int32),
                jax.ShapeDtypeStruct((1, 1), jnp.float32),
            ],
        )(z, cb2, e2)
        accs.append(acc[0, 0])
        idxs.append(idx2d)
        zqps.append(gather(cbp, idx2d))

    zq = jnp.concatenate(zqps, axis=0)[:, :d]
    indices = jnp.concatenate(idxs, axis=0).reshape(n)
    total = accs[0]
    for a in accs[1:]:
        total = total + a
    mean_sq = total / (n * d)
    loss = mean_sq + _BETA * mean_sq
    return (zq, loss, indices)


# speedup vs baseline: 1.1434x; 1.0392x over previous
import jax
import jax.numpy as jnp
from jax import lax
from jax.experimental import pallas as pl
from jax.experimental.pallas import tpu as pltpu
from jax.experimental.pallas import tpu_sc as plsc

_N_E = 1024
_BETA = 0.25
_BLOCK = 2048
_NCHUNKS = 4

_NC = 2
_NS = 16
_NW = _NC * _NS
_CHUNK = 128


def _vq_body(z_ref, cb2_ref, e2_ref, idx_ref, acc_ref):
    i = pl.program_id(0)
    z = z_ref[...]
    cb2 = cb2_ref[...]
    e2 = e2_ref[...]
    z2 = jnp.sum(z * z, axis=1, keepdims=True)
    mm2 = lax.dot_general(z, cb2, (((1,), (1,)), ((), ())),
                          preferred_element_type=jnp.float32)
    dist = (z2 + e2) - mm2
    mind = jnp.min(dist, axis=1, keepdims=True)
    col = lax.broadcasted_iota(jnp.int32, dist.shape, 1)
    idx = jnp.min(jnp.where(dist == mind, col, _N_E), axis=1)
    idx_ref[...] = idx.reshape(idx_ref.shape)

    @pl.when(i == 0)
    def _init():
        acc_ref[...] = jnp.zeros_like(acc_ref)

    acc_ref[...] += jnp.sum(mind).reshape(1, 1)


def _transpose_body(zqp_ref, _zqt_in_ref, zqt_ref):
    zqt_ref[...] = jnp.transpose(zqp_ref[...][:, :64], (1, 0))


def _gather_body(cbp_hbm, idx_hbm, out_hbm, idx_v, rows_v, sem):
    wid = lax.axis_index("s") * _NC + lax.axis_index("c")
    nch = idx_hbm.shape[0] // _NW
    bpw = nch * _CHUNK
    pltpu.sync_copy(idx_hbm.at[pl.ds(wid * nch, nch)], idx_v)
    copies = [pltpu.async_copy(cbp_hbm.at[idx_v.at[j]],
                               rows_v.at[pl.ds(j * _CHUNK, _CHUNK)], sem)
              for j in range(nch)]
    for c in copies:
        c.wait()
    pltpu.sync_copy(rows_v, out_hbm.at[pl.ds(wid * bpw, bpw)])


@jax.jit
def kernel(z, codebook):
    n, d = z.shape
    n_e = codebook.shape[0]
    nrow = _BLOCK // _CHUNK
    nb = n // _BLOCK
    nbc = nb // _NCHUNKS
    nc = n // _NCHUNKS
    nch = nc // _NW // _CHUNK

    cb2 = codebook + codebook
    e2 = jnp.sum(codebook ** 2, axis=1).reshape(1, n_e)
    cbp = jnp.concatenate(
        [codebook, jnp.zeros((n_e, 128 - d), jnp.float32)], axis=1)
    mesh = plsc.VectorSubcoreMesh(core_axis_name="c", subcore_axis_name="s")

    gather = pl.kernel(
        _gather_body,
        mesh=mesh,
        out_type=jax.ShapeDtypeStruct((nc, 128), jnp.float32),
        scratch_types=[
            pltpu.VMEM((nch, _CHUNK), jnp.int32),
            pltpu.VMEM((nch * _CHUNK, 128), jnp.float32),
            pltpu.SemaphoreType.DMA,
        ],
    )

    tb = 2048
    zqt = jnp.zeros((d, n), jnp.float32)
    idxs, accs = [], []
    for c in range(_NCHUNKS):
        base = c * nbc
        idx2d, acc = pl.pallas_call(
            _vq_body,
            grid=(nbc,),
            in_specs=[
                pl.BlockSpec((_BLOCK, d), lambda i, b=base: (b + i, 0)),
                pl.BlockSpec((n_e, d), lambda i: (0, 0)),
                pl.BlockSpec((1, n_e), lambda i: (0, 0)),
            ],
            out_specs=[
                pl.BlockSpec((nrow, _CHUNK), lambda i: (i, 0)),
                pl.BlockSpec((1, 1), lambda i: (0, 0)),
            ],
            out_shape=[
                jax.ShapeDtypeStruct((nc // _CHUNK, _CHUNK), jnp.int32),
                jax.ShapeDtypeStruct((1, 1), jnp.float32),
            ],
        )(z, cb2, e2)
        accs.append(acc[0, 0])
        idxs.append(idx2d)
        zqp = gather(cbp, idx2d)
        col_base = c * (nc // tb)
        zqt = pl.pallas_call(
            _transpose_body,
            grid=(nc // tb,),
            in_specs=[
                pl.BlockSpec((tb, 128), lambda i: (i, 0)),
                pl.BlockSpec(memory_space=pltpu.MemorySpace.HBM),
            ],
            out_specs=pl.BlockSpec((d, tb), lambda i, b=col_base: (0, b + i)),
            out_shape=jax.ShapeDtypeStruct((d, n), jnp.float32),
            input_output_aliases={1: 0},
        )(zqp, zqt)

    zq = zqt.T
    indices = jnp.concatenate(idxs, axis=0).reshape(n)
    total = accs[0]
    for a in accs[1:]:
        total = total + a
    mean_sq = total / (n * d)
    loss = mean_sq + _BETA * mean_sq
    return (zq, loss, indices)


# speedup vs baseline: 1.3394x; 1.1715x over previous
import jax
import jax.numpy as jnp
from jax import lax
from jax.experimental import pallas as pl
from jax.experimental.pallas import tpu as pltpu
from jax.experimental.pallas import tpu_sc as plsc

_N_E = 1024
_BETA = 0.25
_BLOCK = 2048
_NCHUNKS = 4

_NC = 2
_NS = 16
_NW = _NC * _NS
_CHUNK = 128


def _vq_body(zt_ref, cb2_ref, e2_ref, z2_ref, idx_ref, acc_ref):
    i = pl.program_id(0)
    zt = zt_ref[...]
    cb2 = cb2_ref[...]
    e2 = e2_ref[...]
    z2 = z2_ref[...]
    mm2 = lax.dot_general(cb2, zt, (((1,), (0,)), ((), ())),
                          preferred_element_type=jnp.float32)
    dist = (z2 + e2) - mm2
    mind = jnp.min(dist, axis=0, keepdims=True)
    row = lax.broadcasted_iota(jnp.int32, dist.shape, 0)
    idx = jnp.min(jnp.where(dist == mind, row, _N_E), axis=0)
    idx_ref[...] = idx.reshape(idx_ref.shape)

    @pl.when(i == 0)
    def _init():
        acc_ref[...] = jnp.zeros_like(acc_ref)

    acc_ref[...] += jnp.sum(mind).reshape(1, 1)


def _transpose_body(zqp_ref, _zqt_in_ref, zqt_ref):
    zqt_ref[...] = jnp.transpose(zqp_ref[...][:, :64], (1, 0))


def _gather_body(cbp_hbm, idx_hbm, out_hbm, idx_v, rows_v, sem):
    wid = lax.axis_index("s") * _NC + lax.axis_index("c")
    nch = idx_hbm.shape[0] // _NW
    bpw = nch * _CHUNK
    pltpu.sync_copy(idx_hbm.at[pl.ds(wid * nch, nch)], idx_v)
    copies = [pltpu.async_copy(cbp_hbm.at[idx_v.at[j]],
                               rows_v.at[pl.ds(j * _CHUNK, _CHUNK)], sem)
              for j in range(nch)]
    for c in copies:
        c.wait()
    pltpu.sync_copy(rows_v, out_hbm.at[pl.ds(wid * bpw, bpw)])


@jax.jit
def kernel(z, codebook):
    n, d = z.shape
    n_e = codebook.shape[0]
    nrow = _BLOCK // _CHUNK
    nb = n // _BLOCK
    nbc = nb // _NCHUNKS
    nc = n // _NCHUNKS
    nch = nc // _NW // _CHUNK

    cb2 = codebook + codebook
    e2 = jnp.sum(codebook ** 2, axis=1).reshape(n_e, 1)
    z2 = jnp.sum(z ** 2, axis=1).reshape(1, n)
    zt = z.T
    cbp = jnp.concatenate(
        [codebook, jnp.zeros((n_e, 128 - d), jnp.float32)], axis=1)
    mesh = plsc.VectorSubcoreMesh(core_axis_name="c", subcore_axis_name="s")

    gather = pl.kernel(
        _gather_body,
        mesh=mesh,
        out_type=jax.ShapeDtypeStruct((nc, 128), jnp.float32),
        scratch_types=[
            pltpu.VMEM((nch, _CHUNK), jnp.int32),
            pltpu.VMEM((nch * _CHUNK, 128), jnp.float32),
            pltpu.SemaphoreType.DMA,
        ],
    )

    tb = 2048
    zqt = jnp.zeros((d, n), jnp.float32)
    idxs, accs = [], []
    for c in range(_NCHUNKS):
        base = c * nbc
        idx2d, acc = pl.pallas_call(
            _vq_body,
            grid=(nbc,),
            in_specs=[
                pl.BlockSpec((d, _BLOCK), lambda i, b=base: (0, b + i)),
                pl.BlockSpec((n_e, d), lambda i: (0, 0)),
                pl.BlockSpec((n_e, 1), lambda i: (0, 0)),
                pl.BlockSpec((1, _BLOCK), lambda i, b=base: (0, b + i)),
            ],
            out_specs=[
                pl.BlockSpec((nrow, _CHUNK), lambda i: (i, 0)),
                pl.BlockSpec((1, 1), lambda i: (0, 0)),
            ],
            out_shape=[
                jax.ShapeDtypeStruct((nc // _CHUNK, _CHUNK), jnp.int32),
                jax.ShapeDtypeStruct((1, 1), jnp.float32),
            ],
        )(zt, cb2, e2, z2)
        accs.append(acc[0, 0])
        idxs.append(idx2d)
        zqp = gather(cbp, idx2d)
        col_base = c * (nc // tb)
        zqt = pl.pallas_call(
            _transpose_body,
            grid=(nc // tb,),
            in_specs=[
                pl.BlockSpec((tb, 128), lambda i: (i, 0)),
                pl.BlockSpec(memory_space=pltpu.MemorySpace.HBM),
            ],
            out_specs=pl.BlockSpec((d, tb), lambda i, b=col_base: (0, b + i)),
            out_shape=jax.ShapeDtypeStruct((d, n), jnp.float32),
            input_output_aliases={1: 0},
        )(zqp, zqt)

    zq = zqt.T
    indices = jnp.concatenate(idxs, axis=0).reshape(n)
    total = accs[0]
    for a in accs[1:]:
        total = total + a
    mean_sq = total / (n * d)
    loss = mean_sq + _BETA * mean_sq
    return (zq, loss, indices)


# speedup vs baseline: 1.3825x; 1.0321x over previous
import jax
import jax.numpy as jnp
from jax import lax
from jax.experimental import pallas as pl
from jax.experimental.pallas import tpu as pltpu
from jax.experimental.pallas import tpu_sc as plsc

_N_E = 1024
_BETA = 0.25
_BLOCK = 2048
_NCHUNKS = 4

_NC = 2
_NS = 16
_NW = _NC * _NS
_CHUNK = 128


def _vq_body(zt_ref, cb2_ref, e2_ref, z2_ref, idx_ref, acc_ref):
    i = pl.program_id(0)
    zt = zt_ref[...]
    cb2 = cb2_ref[...]
    e2 = e2_ref[...]
    z2 = z2_ref[...]
    mm2 = lax.dot_general(cb2, zt, (((1,), (0,)), ((), ())),
                          preferred_element_type=jnp.float32)
    dist = (z2 + e2) - mm2
    mind = jnp.min(dist, axis=0, keepdims=True)
    row = lax.broadcasted_iota(jnp.int32, dist.shape, 0)
    idx = jnp.min(jnp.where(dist == mind, row, _N_E), axis=0)
    idx_ref[...] = idx.reshape(idx_ref.shape)

    @pl.when(i == 0)
    def _init():
        acc_ref[...] = jnp.zeros_like(acc_ref)

    acc_ref[...] += jnp.sum(mind).reshape(1, 1)


def _transpose0_body(zqp_ref, zqt_ref):
    zqt_ref[...] = jnp.transpose(zqp_ref[...][:, :64], (1, 0))


def _transpose_body(zqp_ref, _zqt_in_ref, zqt_ref):
    zqt_ref[...] = jnp.transpose(zqp_ref[...][:, :64], (1, 0))


def _gather_body(cbp_hbm, idx_hbm, out_hbm, idx_v, rows_v, sem):
    wid = lax.axis_index("s") * _NC + lax.axis_index("c")
    nch = idx_hbm.shape[0] // _NW
    bpw = nch * _CHUNK
    pltpu.sync_copy(idx_hbm.at[pl.ds(wid * nch, nch)], idx_v)
    copies = [pltpu.async_copy(cbp_hbm.at[idx_v.at[j]],
                               rows_v.at[pl.ds(j * _CHUNK, _CHUNK)], sem)
              for j in range(nch)]
    for c in copies:
        c.wait()
    pltpu.sync_copy(rows_v, out_hbm.at[pl.ds(wid * bpw, bpw)])


@jax.jit
def kernel(z, codebook):
    n, d = z.shape
    n_e = codebook.shape[0]
    nrow = _BLOCK // _CHUNK
    nb = n // _BLOCK
    nbc = nb // _NCHUNKS
    nc = n // _NCHUNKS
    nch = nc // _NW // _CHUNK

    cb2 = codebook + codebook
    e2 = jnp.sum(codebook ** 2, axis=1).reshape(n_e, 1)
    z2 = jnp.sum(z ** 2, axis=1).reshape(1, n)
    zt = z.T
    cbp = jnp.concatenate(
        [codebook, jnp.zeros((n_e, 128 - d), jnp.float32)], axis=1)
    mesh = plsc.VectorSubcoreMesh(core_axis_name="c", subcore_axis_name="s")

    gather = pl.kernel(
        _gather_body,
        mesh=mesh,
        out_type=jax.ShapeDtypeStruct((nc, 128), jnp.float32),
        scratch_types=[
            pltpu.VMEM((nch, _CHUNK), jnp.int32),
            pltpu.VMEM((nch * _CHUNK, 128), jnp.float32),
            pltpu.SemaphoreType.DMA,
        ],
    )

    tb = 2048
    zqt = None
    idxs, accs = [], []
    for c in range(_NCHUNKS):
        base = c * nbc
        idx2d, acc = pl.pallas_call(
            _vq_body,
            grid=(nbc,),
            in_specs=[
                pl.BlockSpec((d, _BLOCK), lambda i, b=base: (0, b + i)),
                pl.BlockSpec((n_e, d), lambda i: (0, 0)),
                pl.BlockSpec((n_e, 1), lambda i: (0, 0)),
                pl.BlockSpec((1, _BLOCK), lambda i, b=base: (0, b + i)),
            ],
            out_specs=[
                pl.BlockSpec((nrow, _CHUNK), lambda i: (i, 0)),
                pl.BlockSpec((1, 1), lambda i: (0, 0)),
            ],
            out_shape=[
                jax.ShapeDtypeStruct((nc // _CHUNK, _CHUNK), jnp.int32),
                jax.ShapeDtypeStruct((1, 1), jnp.float32),
            ],
        )(zt, cb2, e2, z2)
        accs.append(acc[0, 0])
        idxs.append(idx2d)
        zqp = gather(cbp, idx2d)
        col_base = c * (nc // tb)
        if zqt is None:
            zqt = pl.pallas_call(
                _transpose0_body,
                grid=(nc // tb,),
                in_specs=[pl.BlockSpec((tb, 128), lambda i: (i, 0))],
                out_specs=pl.BlockSpec((d, tb), lambda i: (0, i)),
                out_shape=jax.ShapeDtypeStruct((d, n), jnp.float32),
            )(zqp)
        else:
            zqt = pl.pallas_call(
                _transpose_body,
                grid=(nc // tb,),
                in_specs=[
                    pl.BlockSpec((tb, 128), lambda i: (i, 0)),
                    pl.BlockSpec(memory_space=pltpu.MemorySpace.HBM),
                ],
                out_specs=pl.BlockSpec((d, tb), lambda i, b=col_base: (0, b + i)),
                out_shape=jax.ShapeDtypeStruct((d, n), jnp.float32),
                input_output_aliases={1: 0},
            )(zqp, zqt)

    zq = zqt.T
    indices = jnp.concatenate(idxs, axis=0).reshape(n)
    total = accs[0]
    for a in accs[1:]:
        total = total + a
    mean_sq = total / (n * d)
    loss = mean_sq + _BETA * mean_sq
    return (zq, loss, indices)


# speedup vs baseline: 1.4795x; 1.0702x over previous
import jax
import jax.numpy as jnp
from jax import lax
from jax.experimental import pallas as pl
from jax.experimental.pallas import tpu as pltpu
from jax.experimental.pallas import tpu_sc as plsc

_N_E = 1024
_BETA = 0.25
_BLOCK = 2048
_NCHUNKS = 4

_NC = 2
_NS = 16
_NW = _NC * _NS
_CHUNK = 128


def _vq_t0_body(zt_ref, cb2_ref, e2_ref, z2_ref, zqp_ref,
                idx_ref, acc_ref, zqt_ref):
    zqt_ref[...] = jnp.transpose(zqp_ref[...][:, :64], (1, 0))
    _vq_body(zt_ref, cb2_ref, e2_ref, z2_ref, idx_ref, acc_ref)


def _vq_t_body(zt_ref, cb2_ref, e2_ref, z2_ref, zqp_ref, _zqt_in_ref,
               idx_ref, acc_ref, zqt_ref):
    zqt_ref[...] = jnp.transpose(zqp_ref[...][:, :64], (1, 0))
    _vq_body(zt_ref, cb2_ref, e2_ref, z2_ref, idx_ref, acc_ref)


def _vq_body(zt_ref, cb2_ref, e2_ref, z2_ref, idx_ref, acc_ref):
    i = pl.program_id(0)
    zt = zt_ref[...]
    cb2 = cb2_ref[...]
    e2 = e2_ref[...]
    z2 = z2_ref[...]
    mm2 = lax.dot_general(cb2, zt, (((1,), (0,)), ((), ())),
                          preferred_element_type=jnp.float32)
    dist = (z2 + e2) - mm2
    mind = jnp.min(dist, axis=0, keepdims=True)
    row = lax.broadcasted_iota(jnp.int32, dist.shape, 0)
    idx = jnp.min(jnp.where(dist == mind, row, _N_E), axis=0)
    idx_ref[...] = idx.reshape(idx_ref.shape)

    @pl.when(i == 0)
    def _init():
        acc_ref[...] = jnp.zeros_like(acc_ref)

    acc_ref[...] += jnp.sum(mind).reshape(1, 1)


def _transpose0_body(zqp_ref, zqt_ref):
    zqt_ref[...] = jnp.transpose(zqp_ref[...][:, :64], (1, 0))


def _transpose_body(zqp_ref, _zqt_in_ref, zqt_ref):
    zqt_ref[...] = jnp.transpose(zqp_ref[...][:, :64], (1, 0))


def _gather_body(cbp_hbm, idx_hbm, out_hbm, idx_v, rows_v, sem):
    wid = lax.axis_index("s") * _NC + lax.axis_index("c")
    nch = idx_hbm.shape[0] // _NW
    bpw = nch * _CHUNK
    pltpu.sync_copy(idx_hbm.at[pl.ds(wid * nch, nch)], idx_v)
    copies = [pltpu.async_copy(cbp_hbm.at[idx_v.at[j]],
                               rows_v.at[pl.ds(j * _CHUNK, _CHUNK)], sem)
              for j in range(nch)]
    for c in copies:
        c.wait()
    pltpu.sync_copy(rows_v, out_hbm.at[pl.ds(wid * bpw, bpw)])


@jax.jit
def kernel(z, codebook):
    n, d = z.shape
    n_e = codebook.shape[0]
    nrow = _BLOCK // _CHUNK
    nb = n // _BLOCK
    nbc = nb // _NCHUNKS
    nc = n // _NCHUNKS
    nch = nc // _NW // _CHUNK

    cb2 = codebook + codebook
    e2 = jnp.sum(codebook ** 2, axis=1).reshape(n_e, 1)
    z2 = jnp.sum(z ** 2, axis=1).reshape(1, n)
    zt = z.T
    cbp = jnp.concatenate(
        [codebook, jnp.zeros((n_e, 128 - d), jnp.float32)], axis=1)
    mesh = plsc.VectorSubcoreMesh(core_axis_name="c", subcore_axis_name="s")

    gather = pl.kernel(
        _gather_body,
        mesh=mesh,
        out_type=jax.ShapeDtypeStruct((nc, 128), jnp.float32),
        scratch_types=[
            pltpu.VMEM((nch, _CHUNK), jnp.int32),
            pltpu.VMEM((nch * _CHUNK, 128), jnp.float32),
            pltpu.SemaphoreType.DMA,
        ],
    )

    tb = 2048
    zqt = None
    zqps, idxs, accs = [], [], []
    vq_in_specs = [
        pl.BlockSpec((d, _BLOCK), lambda i, b=0: (0, b + i)),
        pl.BlockSpec((n_e, d), lambda i: (0, 0)),
        pl.BlockSpec((n_e, 1), lambda i: (0, 0)),
        pl.BlockSpec((1, _BLOCK), lambda i, b=0: (0, b + i)),
    ]
    vq_out_specs = [
        pl.BlockSpec((nrow, _CHUNK), lambda i: (i, 0)),
        pl.BlockSpec((1, 1), lambda i: (0, 0)),
    ]
    vq_out_shape = [
        jax.ShapeDtypeStruct((nc // _CHUNK, _CHUNK), jnp.int32),
        jax.ShapeDtypeStruct((1, 1), jnp.float32),
    ]
    for c in range(_NCHUNKS):
        base = c * nbc
        in_specs = [
            pl.BlockSpec((d, _BLOCK), lambda i, b=base: (0, b + i)),
            vq_in_specs[1],
            vq_in_specs[2],
            pl.BlockSpec((1, _BLOCK), lambda i, b=base: (0, b + i)),
        ]
        if c < 2:
            idx2d, acc = pl.pallas_call(
                _vq_body,
                grid=(nbc,),
                in_specs=in_specs,
                out_specs=vq_out_specs,
                out_shape=vq_out_shape,
            )(zt, cb2, e2, z2)
        else:
            tcol = (c - 2) * (nc // tb)
            tspec = pl.BlockSpec((tb, 128), lambda i: (i, 0))
            ospec = pl.BlockSpec((d, tb), lambda i, b=tcol: (0, b + i))
            oshape = jax.ShapeDtypeStruct((d, n), jnp.float32)
            if zqt is None:
                idx2d, acc, zqt = pl.pallas_call(
                    _vq_t0_body,
                    grid=(nbc,),
                    in_specs=in_specs + [tspec],
                    out_specs=vq_out_specs + [ospec],
                    out_shape=vq_out_shape + [oshape],
                )(zt, cb2, e2, z2, zqps[c - 2])
            else:
                idx2d, acc, zqt = pl.pallas_call(
                    _vq_t_body,
                    grid=(nbc,),
                    in_specs=in_specs + [
                        tspec, pl.BlockSpec(memory_space=pltpu.MemorySpace.HBM)],
                    out_specs=vq_out_specs + [ospec],
                    out_shape=vq_out_shape + [oshape],
                    input_output_aliases={5: 2},
                )(zt, cb2, e2, z2, zqps[c - 2], zqt)
        accs.append(acc[0, 0])
        idxs.append(idx2d)
        zqps.append(gather(cbp, idx2d))

    for c in range(_NCHUNKS - 2, _NCHUNKS):
        col_base = c * (nc // tb)
        zqt = pl.pallas_call(
            _transpose_body,
            grid=(nc // tb,),
            in_specs=[
                pl.BlockSpec((tb, 128), lambda i: (i, 0)),
                pl.BlockSpec(memory_space=pltpu.MemorySpace.HBM),
            ],
            out_specs=pl.BlockSpec((d, tb), lambda i, b=col_base: (0, b + i)),
            out_shape=jax.ShapeDtypeStruct((d, n), jnp.float32),
            input_output_aliases={1: 0},
        )(zqps[c], zqt)

    zq = zqt.T
    indices = jnp.concatenate(idxs, axis=0).reshape(n)
    total = accs[0]
    for a in accs[1:]:
        total = total + a
    mean_sq = total / (n * d)
    loss = mean_sq + _BETA * mean_sq
    return (zq, loss, indices)


# speedup vs baseline: 1.4817x; 1.0014x over previous
import jax
import jax.numpy as jnp
from jax import lax
from jax.experimental import pallas as pl
from jax.experimental.pallas import tpu as pltpu
from jax.experimental.pallas import tpu_sc as plsc

_N_E = 1024
_BETA = 0.25
_BLOCK = 2048
_NCHUNKS = 4

_NC = 2
_NS = 16
_NW = _NC * _NS
_CHUNK = 128


def _vq_t0_body(zt_ref, cb2_ref, e2_ref, z2_ref, zqp_ref,
                idx_ref, acc_ref, zqt_ref):
    zqt_ref[...] = jnp.transpose(zqp_ref[...][:, :64], (1, 0))
    _vq_body(zt_ref, cb2_ref, e2_ref, z2_ref, idx_ref, acc_ref)


def _vq_t_body(zt_ref, cb2_ref, e2_ref, z2_ref, zqp_ref, _zqt_in_ref,
               idx_ref, acc_ref, zqt_ref):
    zqt_ref[...] = jnp.transpose(zqp_ref[...][:, :64], (1, 0))
    _vq_body(zt_ref, cb2_ref, e2_ref, z2_ref, idx_ref, acc_ref)


def _vq_body(zt_ref, cb2_ref, e2_ref, z2_ref, idx_ref, acc_ref):
    i = pl.program_id(0)
    zt = zt_ref[...]
    cb2 = cb2_ref[...]
    e2 = e2_ref[...]
    z2 = z2_ref[...]
    mm2 = lax.dot_general(cb2, zt, (((1,), (0,)), ((), ())),
                          preferred_element_type=jnp.float32)
    dist = (z2 + e2) - mm2
    mind = jnp.min(dist, axis=0, keepdims=True)
    row = lax.broadcasted_iota(jnp.int32, dist.shape, 0)
    idx = jnp.min(jnp.where(dist == mind, row, _N_E), axis=0)
    idx_ref[...] = idx.reshape(idx_ref.shape)

    @pl.when(i == 0)
    def _init():
        acc_ref[...] = jnp.zeros_like(acc_ref)

    acc_ref[...] += jnp.sum(mind).reshape(1, 1)


def _transpose_body(zqp_ref, _zqt_in_ref, zqt_ref):
    zqt_ref[...] = jnp.transpose(zqp_ref[...][:, :64], (1, 0))


def _gather_body(cbp_hbm, idx_hbm, out_hbm, idx_v, rows_v, sem):
    wid = lax.axis_index("s") * _NC + lax.axis_index("c")
    nch = idx_hbm.shape[0] // _NW
    bpw = nch * _CHUNK
    pltpu.sync_copy(idx_hbm.at[pl.ds(wid * nch, nch)], idx_v)
    copies = [pltpu.async_copy(cbp_hbm.at[idx_v.at[j]],
                               rows_v.at[pl.ds(j * _CHUNK, _CHUNK)], sem)
              for j in range(nch)]
    for c in copies:
        c.wait()
    pltpu.sync_copy(rows_v, out_hbm.at[pl.ds(wid * bpw, bpw)])


@jax.jit
def kernel(z, codebook):
    n, d = z.shape
    n_e = codebook.shape[0]
    nrow = _BLOCK // _CHUNK
    nb = n // _BLOCK
    nbc = nb // _NCHUNKS
    nc = n // _NCHUNKS
    nch = nc // _NW // _CHUNK

    cb2 = codebook + codebook
    e2 = jnp.sum(codebook ** 2, axis=1).reshape(n_e, 1)
    z2 = jnp.sum(z ** 2, axis=1).reshape(1, n)
    zt = z.T
    cbp = jnp.concatenate(
        [codebook, jnp.zeros((n_e, 128 - d), jnp.float32)], axis=1)
    mesh = plsc.VectorSubcoreMesh(core_axis_name="c", subcore_axis_name="s")

    gather = pl.kernel(
        _gather_body,
        mesh=mesh,
        out_type=jax.ShapeDtypeStruct((nc, 128), jnp.float32),
        scratch_types=[
            pltpu.VMEM((nch, _CHUNK), jnp.int32),
            pltpu.VMEM((nch * _CHUNK, 128), jnp.float32),
            pltpu.SemaphoreType.DMA,
        ],
    )

    tb = 2048
    zqt = None
    zqps, idxs, accs = [], [], []
    cb2_spec = pl.BlockSpec((n_e, d), lambda i: (0, 0))
    e2_spec = pl.BlockSpec((n_e, 1), lambda i: (0, 0))
    vq_out_specs = [
        pl.BlockSpec((nrow, _CHUNK), lambda i: (i, 0)),
        pl.BlockSpec((1, 1), lambda i: (0, 0)),
    ]
    vq_out_shape = [
        jax.ShapeDtypeStruct((nc // _CHUNK, _CHUNK), jnp.int32),
        jax.ShapeDtypeStruct((1, 1), jnp.float32),
    ]
    for c in range(_NCHUNKS):
        base = c * nbc
        in_specs = [
            pl.BlockSpec((d, _BLOCK), lambda i, b=base: (0, b + i)),
            cb2_spec,
            e2_spec,
            pl.BlockSpec((1, _BLOCK), lambda i, b=base: (0, b + i)),
        ]
        if c < 2:
            idx2d, acc = pl.pallas_call(
                _vq_body,
                grid=(nbc,),
                in_specs=in_specs,
                out_specs=vq_out_specs,
                out_shape=vq_out_shape,
            )(zt, cb2, e2, z2)
        else:
            tcol = (c - 2) * (nc // tb)
            tspec = pl.BlockSpec((tb, 128), lambda i: (i, 0))
            ospec = pl.BlockSpec((d, tb), lambda i, b=tcol: (0, b + i))
            oshape = jax.ShapeDtypeStruct((d, n), jnp.float32)
            if zqt is None:
                idx2d, acc, zqt = pl.pallas_call(
                    _vq_t0_body,
                    grid=(nbc,),
                    in_specs=in_specs + [tspec],
                    out_specs=vq_out_specs + [ospec],
                    out_shape=vq_out_shape + [oshape],
                )(zt, cb2, e2, z2, zqps[c - 2])
            else:
                idx2d, acc, zqt = pl.pallas_call(
                    _vq_t_body,
                    grid=(nbc,),
                    in_specs=in_specs + [
                        tspec, pl.BlockSpec(memory_space=pltpu.MemorySpace.HBM)],
                    out_specs=vq_out_specs + [ospec],
                    out_shape=vq_out_shape + [oshape],
                    input_output_aliases={5: 2},
                )(zt, cb2, e2, z2, zqps[c - 2], zqt)
        accs.append(acc[0, 0])
        idxs.append(idx2d)
        zqps.append(gather(cbp, idx2d))

    for c in range(_NCHUNKS - 2, _NCHUNKS):
        col_base = c * (nc // tb)
        zqt = pl.pallas_call(
            _transpose_body,
            grid=(nc // tb,),
            in_specs=[
                pl.BlockSpec((tb, 128), lambda i: (i, 0)),
                pl.BlockSpec(memory_space=pltpu.MemorySpace.HBM),
            ],
            out_specs=pl.BlockSpec((d, tb), lambda i, b=col_base: (0, b + i)),
            out_shape=jax.ShapeDtypeStruct((d, n), jnp.float32),
            input_output_aliases={1: 0},
        )(zqps[c], zqt)

    zq = zqt.T
    indices = jnp.concatenate(idxs, axis=0).reshape(n)
    total = accs[0]
    for a in accs[1:]:
        total = total + a
    mean_sq = total / (n * d)
    loss = mean_sq + _BETA * mean_sq
    return (zq, loss, indices)
